# baseline (speedup 1.0000x reference)
import jax
import jax.numpy as jnp
from jax.experimental import pallas as pl
from jax.experimental.pallas import tpu as pltpu

_N_GRAPHS = 256
_N_ACT = 8


def _bn(h, g, b):
    m = jnp.mean(h, axis=0, keepdims=True)
    v = jnp.var(h, axis=0, keepdims=True)
    return g * (h - m) / jnp.sqrt(v + 1e-5) + b


def _gcn(h, src, dst, ew, W, b, n):
    h = h @ W
    loop = jnp.arange(n, dtype=src.dtype)
    s = jnp.concatenate([src, loop])
    d = jnp.concatenate([dst, loop])
    w = jnp.concatenate([ew, jnp.ones((n,), h.dtype)])
    deg = jax.ops.segment_sum(w, d, num_segments=n)
    dinv = jnp.power(jnp.maximum(deg, 1e-12), -0.5)
    norm = dinv[s] * w * dinv[d]
    return jax.ops.segment_sum(norm[:, None] * h[s], d, num_segments=n) + b


def _head_kernel(hh_ref, w0_ref, b0_ref, g_ref, be_ref, w1_ref, b1_ref, o_ref):
    hh = hh_ref[...]
    y = hh @ w0_ref[...] + b0_ref[...]
    m = jnp.mean(y, axis=0, keepdims=True)
    v = jnp.mean((y - m) * (y - m), axis=0, keepdims=True)
    y = jax.nn.relu(g_ref[...] * (y - m) / jnp.sqrt(v + 1e-5) + be_ref[...])
    o_ref[...] = y @ w1_ref[...] + b1_ref[...]


def kernel(x, edge_weight, speed, route, params, edge_index, batch_ids):
    p = params
    src, dst = edge_index[0], edge_index[1]
    n = x.shape[0]
    h = jax.nn.relu(_bn(x @ p['W_emb'] + p['b_emb'], p['g_emb'], p['be_emb']))
    h = jax.nn.relu(_bn(_gcn(h, src, dst, edge_weight, p['W_c0'], p['b_c0'], n), p['g_bn0'], p['be_bn0']))
    h = jax.nn.relu(_bn(_gcn(h, src, dst, edge_weight, p['W_c1'], p['b_c1'], n), p['g_bn1'], p['be_bn1']))
    counts = jax.ops.segment_sum(jnp.ones((n,), h.dtype), batch_ids, num_segments=_N_GRAPHS)
    pooled = jax.ops.segment_sum(h, batch_ids, num_segments=_N_GRAPHS) / jnp.maximum(counts, 1.0)[:, None]
    v = jax.nn.relu(_bn(speed @ p['W_sp'] + p['b_sp'], p['g_sp'], p['be_sp']))
    rp = jnp.transpose(route, (0, 2, 1))
    rc = jax.lax.conv_general_dilated(rp, p['W_rc'], window_strides=(1,), padding=((1, 1),),
                                      dimension_numbers=('NCH', 'OIH', 'NCH')) + p['b_rc'][None, :, None]
    m = jnp.mean(rc, axis=(0, 2), keepdims=True)
    var = jnp.var(rc, axis=(0, 2), keepdims=True)
    rc = jax.nn.relu(p['g_rc'][None, :, None] * (rc - m) / jnp.sqrt(var + 1e-5) + p['be_rc'][None, :, None])
    r = rc.reshape(rc.shape[0], -1) @ p['W_rl'] + p['b_rl']
    hh = jnp.concatenate([pooled, v, r], axis=1)
    out = pl.pallas_call(
        _head_kernel,
        out_shape=jax.ShapeDtypeStruct((_N_GRAPHS, _N_ACT), jnp.float32),
    )(hh, p['W_o0'], p['b_o0'], p['g_o'], p['be_o'], p['W_o1'], p['b_o1'])
    return jnp.squeeze(out)


# speedup vs baseline: 28.5735x; 28.5735x over previous
import functools

import jax
import jax.numpy as jnp
from jax import lax
from jax.experimental import pallas as pl
from jax.experimental.pallas import tpu as pltpu
from jax.experimental.pallas import tpu_sc as plsc

N = 10000
NP = 10240
E = 320000
F = 128
H = 16
G = 256
NW = 32
EPW = E // NW
CH = 2000
NCH = EPW // CH
PT = NP // 16


def _deg_body(dst_hbm, w_hbm, out_hbm, hist, dst_v, w_v, degp):
    core = lax.axis_index("c")
    sid = lax.axis_index("s")
    wid = core * 16 + sid
    base = wid * EPW

    zero16 = jnp.zeros((16,), jnp.float32)

    def z(i, _):
        hist[pl.ds(i * 16, 16)] = zero16
        return 0
    lax.fori_loop(0, 8 * NP // 16, z, 0)

    pltpu.sync_copy(dst_hbm.at[pl.ds(base, EPW)], dst_v)
    pltpu.sync_copy(w_hbm.at[pl.ds(base, EPW)], w_v)

    iota = lax.iota(jnp.int32, 16)
    laneoff = (iota & 7) * NP
    m_lo = iota < 8
    m_hi = iota >= 8

    def acc(g, _):
        d16 = dst_v[pl.ds(g * 16, 16)]
        w16 = w_v[pl.ds(g * 16, 16)]
        idx = d16 + laneoff
        plsc.addupdate_scatter(hist, [idx], w16, mask=m_lo)
        plsc.addupdate_scatter(hist, [idx], w16, mask=m_hi)
        return 0
    lax.fori_loop(0, EPW // 16, acc, 0)

    def red(i, _):
        s = hist[pl.ds(i * 16, 16)]
        for l in range(1, 8):
            s = s + hist[pl.ds(l * NP + i * 16, 16)]
        degp[pl.ds(i * 16, 16)] = s
        return 0
    lax.fori_loop(0, NP // 16, red, 0)

    pltpu.sync_copy(degp, out_hbm.at[wid])


_deg_call = functools.partial(
    pl.kernel,
    out_type=jax.ShapeDtypeStruct((NW, NP), jnp.float32),
    scratch_types=[
        pltpu.VMEM((8 * NP,), jnp.float32),
        pltpu.VMEM((EPW,), jnp.int32),
        pltpu.VMEM((EPW,), jnp.float32),
        pltpu.VMEM((NP,), jnp.float32),
    ],
    mesh=plsc.VectorSubcoreMesh(core_axis_name="c", subcore_axis_name="s"),
    compiler_params=pltpu.CompilerParams(needs_layout_passes=False, use_tc_tiling_on_sc=False),
)(_deg_body)


def _agg_body(h_hbm, src_hbm, dst_hbm, w_hbm, dinv_hbm, out_hbm,
              dinv_v, src_v, dst_v, w_v, norm_v, rows_v, acc_sh, sem):
    core = lax.axis_index("c")
    sid = lax.axis_index("s")
    wid = core * 16 + sid
    base = wid * EPW

    pltpu.sync_copy(dinv_hbm, dinv_v)

    zero16 = jnp.zeros((16,), jnp.float32)

    def z(i, _):
        rows_v[i, :] = zero16
        return 0
    lax.fori_loop(0, PT, z, 0)
    pltpu.sync_copy(rows_v.at[pl.ds(0, PT)], acc_sh.at[pl.ds(sid * PT, PT)])
    plsc.subcore_barrier()

    for c in range(NCH):
        eb = base + c * CH
        pltpu.sync_copy(src_hbm.at[pl.ds(eb, CH)], src_v)
        pltpu.sync_copy(dst_hbm.at[pl.ds(eb, CH)], dst_v)
        pltpu.sync_copy(w_hbm.at[pl.ds(eb, CH)], w_v)
        pltpu.async_copy(h_hbm.at[src_v], rows_v, sem).wait()

        def nrm(j, _):
            s16 = src_v[pl.ds(j * 16, 16)]
            d16 = dst_v[pl.ds(j * 16, 16)]
            w16 = w_v[pl.ds(j * 16, 16)]
            nm = plsc.load_gather(dinv_v, [s16]) * w16 * plsc.load_gather(dinv_v, [d16])
            norm_v[pl.ds(j * 16, 16)] = nm
            return 0
        lax.fori_loop(0, CH // 16, nrm, 0)

        def scale(j, _):
            for u in range(16):
                e = j * 16 + u
                nv = plsc.load_gather(norm_v, [jnp.full((16,), 0, jnp.int32) + e])
                rows_v[e, :] = rows_v[e, :] * nv
            return 0
        lax.fori_loop(0, CH // 16, scale, 0)

        pltpu.sync_copy(rows_v, acc_sh.at[dst_v], add=True)

    plsc.subcore_barrier()
    pltpu.sync_copy(acc_sh.at[pl.ds(sid * PT, PT)],
                    out_hbm.at[core, pl.ds(sid * PT, PT)])


_agg_call = functools.partial(
    pl.kernel,
    out_type=jax.ShapeDtypeStruct((2, NP, H), jnp.float32),
    scratch_types=[
        pltpu.VMEM((NP,), jnp.float32),
        pltpu.VMEM((CH,), jnp.int32),
        pltpu.VMEM((CH,), jnp.int32),
        pltpu.VMEM((CH,), jnp.float32),
        pltpu.VMEM((CH,), jnp.float32),
        pltpu.VMEM((CH, H), jnp.float32),
        pltpu.VMEM_SHARED((NP, H), jnp.float32),
        pltpu.SemaphoreType.DMA,
    ],
    mesh=plsc.VectorSubcoreMesh(core_axis_name="c", subcore_axis_name="s"),
    compiler_params=pltpu.CompilerParams(needs_layout_passes=False, use_tc_tiling_on_sc=False),
)(_agg_body)


def _bn_rows(y, g, b):
    m = jnp.mean(y, axis=0, keepdims=True)
    v = jnp.mean((y - m) * (y - m), axis=0, keepdims=True)
    return g * (y - m) * lax.rsqrt(v + 1e-5) + b


def _emb_kernel(x_ref, wemb_ref, bemb_ref, gemb_ref, beemb_ref, wc0_ref,
                degp_ref, h1_ref, dinv_ref):
    y = jnp.dot(x_ref[...], wemb_ref[...], preferred_element_type=jnp.float32)
    y = y + bemb_ref[...]
    h0 = jax.nn.relu(_bn_rows(y, gemb_ref[...], beemb_ref[...]))
    h1_ref[...] = jnp.dot(h0, wc0_ref[...], preferred_element_type=jnp.float32)
    deg = jnp.sum(degp_ref[...], axis=0, keepdims=True) + 1.0
    dinv_ref[...] = lax.rsqrt(jnp.maximum(deg, 1e-12))


def _mid_kernel(a0_ref, a1_ref, hlin_ref, dinv_ref, b_ref, g_ref, be_ref,
                wnext_ref, out_ref):
    d2 = dinv_ref[...] * dinv_ref[...]
    gc = a0_ref[...] + a1_ref[...] + d2 * hlin_ref[...] + b_ref[...]
    h = jax.nn.relu(_bn_rows(gc, g_ref[...], be_ref[...]))
    out_ref[...] = jnp.dot(h, wnext_ref[...], preferred_element_type=jnp.float32)


def _head_kernel(a0_ref, a1_ref, hlin_ref, dinv_ref, bc1_ref, gbn1_ref,
                 bebn1_ref, bid_ref, speed_ref, wsp_ref, bsp_ref, gsp_ref,
                 besp_ref, r0_ref, r1_ref, wrc_ref, brc_ref, grc_ref,
                 berc_ref, wrl_ref, brl_ref, w0p_ref, w0v_ref, w0r_ref,
                 b0_ref, go_ref, beo_ref, w1_ref, b1_ref, out_ref):
    d2 = dinv_ref[...] * dinv_ref[...]
    gc = a0_ref[...] + a1_ref[...] + d2 * hlin_ref[...] + bc1_ref[...]
    h2 = jax.nn.relu(_bn_rows(gc, gbn1_ref[...], bebn1_ref[...]))

    gid = lax.broadcasted_iota(jnp.int32, (G, N), 0)
    onehot = (bid_ref[...] == gid).astype(jnp.float32)
    psum = jnp.dot(onehot, h2, preferred_element_type=jnp.float32)
    counts = jnp.sum(onehot, axis=1, keepdims=True)
    pooled = psum / jnp.maximum(counts, 1.0)

    v = jax.nn.relu(_bn_rows(
        jnp.dot(speed_ref[...], wsp_ref[...], preferred_element_type=jnp.float32)
        + bsp_ref[...], gsp_ref[...], besp_ref[...]))

    zc = jnp.zeros((G, 1), jnp.float32)
    rc = brc_ref[...] * jnp.ones((G, 10), jnp.float32)
    for ci, rr in ((0, r0_ref[...]), (1, r1_ref[...])):
        zp = jnp.concatenate([zc, rr, zc], axis=1)
        for k in range(3):
            rc = rc + wrc_ref[ci:ci + 1, k:k + 1] * zp[:, k:k + 10]
    m = jnp.mean(rc)
    var = jnp.mean((rc - m) * (rc - m))
    rc = jax.nn.relu(grc_ref[...] * (rc - m) * lax.rsqrt(var + 1e-5) + berc_ref[...])
    r = jnp.dot(rc, wrl_ref[...], preferred_element_type=jnp.float32) + brl_ref[...]

    y0 = (jnp.dot(pooled, w0p_ref[...], preferred_element_type=jnp.float32)
          + jnp.dot(v, w0v_ref[...], preferred_element_type=jnp.float32)
          + jnp.dot(r, w0r_ref[...], preferred_element_type=jnp.float32)
          + b0_ref[...])
    y = jax.nn.relu(_bn_rows(y0, go_ref[...], beo_ref[...]))
    out_ref[...] = jnp.dot(y, w1_ref[...], preferred_element_type=jnp.float32) + b1_ref[...]


def _tc(fn, out_shape, *args):
    return pl.pallas_call(fn, out_shape=out_shape)(*args)


def kernel(x, edge_weight, speed, route, params, edge_index, batch_ids):
    p = params
    src = edge_index[0]
    dst = edge_index[1]

    deg_parts = _deg_call(dst, edge_weight)

    h1lin, dinv_row = _tc(
        _emb_kernel,
        (jax.ShapeDtypeStruct((N, H), jnp.float32),
         jax.ShapeDtypeStruct((1, N), jnp.float32)),
        x, p['W_emb'], p['b_emb'].reshape(1, F), p['g_emb'].reshape(1, F),
        p['be_emb'].reshape(1, F), p['W_c0'], deg_parts[:, :N])

    dinv_pad = jnp.pad(dinv_row.reshape(N), (0, NP - N))
    dinv_col = dinv_row.reshape(N, 1)

    acc1 = _agg_call(h1lin, src, dst, edge_weight, dinv_pad)

    h2lin = _tc(
        _mid_kernel, jax.ShapeDtypeStruct((N, H), jnp.float32),
        acc1[0, :N], acc1[1, :N], h1lin, dinv_col, p['b_c0'].reshape(1, H),
        p['g_bn0'].reshape(1, H), p['be_bn0'].reshape(1, H), p['W_c1'])

    acc2 = _agg_call(h2lin, src, dst, edge_weight, dinv_pad)

    out = _tc(
        _head_kernel, jax.ShapeDtypeStruct((G, 8), jnp.float32),
        acc2[0, :N], acc2[1, :N], h2lin, dinv_col, p['b_c1'].reshape(1, H),
        p['g_bn1'].reshape(1, H), p['be_bn1'].reshape(1, H),
        batch_ids.reshape(1, N), speed, p['W_sp'], p['b_sp'].reshape(1, 4),
        p['g_sp'].reshape(1, 4), p['be_sp'].reshape(1, 4),
        route[:, :, 0], route[:, :, 1], p['W_rc'][0], p['b_rc'].reshape(1, 1),
        p['g_rc'].reshape(1, 1), p['be_rc'].reshape(1, 1), p['W_rl'],
        p['b_rl'].reshape(1, 4), p['W_o0'][:H], p['W_o0'][H:H + 4],
        p['W_o0'][H + 4:], p['b_o0'].reshape(1, 16), p['g_o'].reshape(1, 16),
        p['be_o'].reshape(1, 16), p['W_o1'], p['b_o1'].reshape(1, 8))
    return jnp.squeeze(out)


# speedup vs baseline: 31.8707x; 1.1154x over previous
import functools

import jax
import jax.numpy as jnp
from jax import lax
from jax.experimental import pallas as pl
from jax.experimental.pallas import tpu as pltpu
from jax.experimental.pallas import tpu_sc as plsc

N = 10000
NP = 10240
E = 320000
F = 128
H = 16
G = 256
NW = 32
EPW = E // NW
CH = 2000
NCH = EPW // CH
PT = NP // 16


def _deg_body(dst_hbm, w_hbm, out_hbm, hist, dst_v, w_v, degp):
    core = lax.axis_index("c")
    sid = lax.axis_index("s")
    wid = core * 16 + sid
    base = wid * EPW

    zero16 = jnp.zeros((16,), jnp.float32)

    def z(i, _):
        hist[pl.ds(i * 16, 16)] = zero16
        return 0
    lax.fori_loop(0, 8 * NP // 16, z, 0)

    pltpu.sync_copy(dst_hbm.at[pl.ds(base, EPW)], dst_v)
    pltpu.sync_copy(w_hbm.at[pl.ds(base, EPW)], w_v)

    iota = lax.iota(jnp.int32, 16)
    laneoff = (iota & 7) * NP
    m_lo = iota < 8
    m_hi = iota >= 8

    def acc(g, _):
        d16 = dst_v[pl.ds(g * 16, 16)]
        w16 = w_v[pl.ds(g * 16, 16)]
        idx = d16 + laneoff
        plsc.addupdate_scatter(hist, [idx], w16, mask=m_lo)
        plsc.addupdate_scatter(hist, [idx], w16, mask=m_hi)
        return 0
    lax.fori_loop(0, EPW // 16, acc, 0)

    def red(i, _):
        s = hist[pl.ds(i * 16, 16)]
        for l in range(1, 8):
            s = s + hist[pl.ds(l * NP + i * 16, 16)]
        degp[pl.ds(i * 16, 16)] = s
        return 0
    lax.fori_loop(0, NP // 16, red, 0)

    pltpu.sync_copy(degp, out_hbm.at[wid])


_deg_call = functools.partial(
    pl.kernel,
    out_type=jax.ShapeDtypeStruct((NW, NP), jnp.float32),
    scratch_types=[
        pltpu.VMEM((8 * NP,), jnp.float32),
        pltpu.VMEM((EPW,), jnp.int32),
        pltpu.VMEM((EPW,), jnp.float32),
        pltpu.VMEM((NP,), jnp.float32),
    ],
    mesh=plsc.VectorSubcoreMesh(core_axis_name="c", subcore_axis_name="s"),
    compiler_params=pltpu.CompilerParams(needs_layout_passes=False, use_tc_tiling_on_sc=False),
)(_deg_body)


def _rsqrt16(x):
    i = plsc.bitcast(x, jnp.int32)
    y = plsc.bitcast(jnp.full((16,), 0x5F3759DF, jnp.int32) - (i >> 1), jnp.float32)
    xh = x * 0.5
    for _ in range(4):
        y = y * (1.5 - xh * y * y)
    return y


def _agg_body(h_hbm, src_hbm, dst_hbm, w_hbm, degp_hbm, out_hbm, dinv_hbm,
              degs_v, dsl_v, dinv_v, src0, dst0, w0, src1, dst1, w1,
              norm_v, rows0, rows1, acc_sh, dinv_sh,
              gsem0, gsem1, ssem0, ssem1):
    core = lax.axis_index("c")
    sid = lax.axis_index("s")
    wid = core * 16 + sid
    base = wid * EPW

    pltpu.sync_copy(degp_hbm.at[:, pl.ds(sid * PT, PT)], degs_v)

    def dv(i, _):
        s = degs_v[0, pl.ds(i * 16, 16)]
        for l in range(1, NW):
            s = s + degs_v[l, pl.ds(i * 16, 16)]
        dsl_v[pl.ds(i * 16, 16)] = _rsqrt16(s + 1.0)
        return 0
    lax.fori_loop(0, PT // 16, dv, 0)
    pltpu.sync_copy(dsl_v, dinv_sh.at[pl.ds(sid * PT, PT)])

    zero16 = jnp.zeros((16,), jnp.float32)

    def z(i, _):
        rows0[i, :] = zero16
        return 0
    lax.fori_loop(0, PT, z, 0)
    pltpu.sync_copy(rows0.at[pl.ds(0, PT)], acc_sh.at[pl.ds(sid * PT, PT)])
    plsc.subcore_barrier()
    pltpu.sync_copy(dinv_sh, dinv_v)

    @pl.when(jnp.logical_and(core == 0, sid == 0))
    def _():
        pltpu.sync_copy(dinv_v, dinv_hbm)

    bufs = ((src0, dst0, w0, rows0, gsem0, ssem0),
            (src1, dst1, w1, rows1, gsem1, ssem1))

    def stage(c):
        srcb, dstb, wb, rowsb, gsem, _ = bufs[c % 2]
        eb = base + c * CH
        pltpu.sync_copy(src_hbm.at[pl.ds(eb, CH)], srcb)
        pltpu.sync_copy(dst_hbm.at[pl.ds(eb, CH)], dstb)
        pltpu.sync_copy(w_hbm.at[pl.ds(eb, CH)], wb)
        pltpu.async_copy(h_hbm.at[srcb], rowsb, gsem)

    stage(0)
    for c in range(NCH):
        srcb, dstb, wb, rowsb, gsem, ssem = bufs[c % 2]
        pltpu.make_async_copy(h_hbm.at[srcb], rowsb, gsem).wait()
        if c + 1 < NCH:
            if c >= 1:
                pltpu.make_async_copy(
                    bufs[(c + 1) % 2][3], acc_sh.at[bufs[(c + 1) % 2][1]],
                    bufs[(c + 1) % 2][5]).wait()
            stage(c + 1)

        def nrm(j, _):
            s16 = srcb[pl.ds(j * 16, 16)]
            d16 = dstb[pl.ds(j * 16, 16)]
            w16 = wb[pl.ds(j * 16, 16)]
            nm = plsc.load_gather(dinv_v, [s16]) * w16 * plsc.load_gather(dinv_v, [d16])
            norm_v[pl.ds(j * 16, 16)] = nm
            return 0
        lax.fori_loop(0, CH // 16, nrm, 0)

        def scale(j, _):
            for u in range(16):
                e = j * 16 + u
                nv = plsc.load_gather(norm_v, [jnp.broadcast_to(e, (16,))])
                rowsb[e, :] = rowsb[e, :] * nv
            return 0
        lax.fori_loop(0, CH // 16, scale, 0)

        pltpu.async_copy(rowsb, acc_sh.at[dstb], ssem, add=True)

    pltpu.make_async_copy(bufs[(NCH - 2) % 2][3],
                          acc_sh.at[bufs[(NCH - 2) % 2][1]],
                          bufs[(NCH - 2) % 2][5]).wait()
    pltpu.make_async_copy(bufs[(NCH - 1) % 2][3],
                          acc_sh.at[bufs[(NCH - 1) % 2][1]],
                          bufs[(NCH - 1) % 2][5]).wait()
    plsc.subcore_barrier()
    pltpu.sync_copy(acc_sh.at[pl.ds(sid * PT, PT)],
                    out_hbm.at[core, pl.ds(sid * PT, PT)])


_agg_call = functools.partial(
    pl.kernel,
    out_type=(jax.ShapeDtypeStruct((2, NP, H), jnp.float32),
              jax.ShapeDtypeStruct((NP,), jnp.float32)),
    scratch_types=[
        pltpu.VMEM((NW, PT), jnp.float32),
        pltpu.VMEM((PT,), jnp.float32),
        pltpu.VMEM((NP,), jnp.float32),
        pltpu.VMEM((CH,), jnp.int32),
        pltpu.VMEM((CH,), jnp.int32),
        pltpu.VMEM((CH,), jnp.float32),
        pltpu.VMEM((CH,), jnp.int32),
        pltpu.VMEM((CH,), jnp.int32),
        pltpu.VMEM((CH,), jnp.float32),
        pltpu.VMEM((CH,), jnp.float32),
        pltpu.VMEM((CH, H), jnp.float32),
        pltpu.VMEM((CH, H), jnp.float32),
        pltpu.VMEM_SHARED((NP, H), jnp.float32),
        pltpu.VMEM_SHARED((NP,), jnp.float32),
        pltpu.SemaphoreType.DMA,
        pltpu.SemaphoreType.DMA,
        pltpu.SemaphoreType.DMA,
        pltpu.SemaphoreType.DMA,
    ],
    mesh=plsc.VectorSubcoreMesh(core_axis_name="c", subcore_axis_name="s"),
    compiler_params=pltpu.CompilerParams(needs_layout_passes=False, use_tc_tiling_on_sc=False),
)(_agg_body)


def _bn_rows(y, g, b):
    m = jnp.mean(y, axis=0, keepdims=True)
    v = jnp.mean((y - m) * (y - m), axis=0, keepdims=True)
    return g * (y - m) * lax.rsqrt(v + 1e-5) + b


def _emb_kernel(x_ref, wemb_ref, bemb_ref, gemb_ref, beemb_ref, wc0_ref,
                h1_ref):
    y = jnp.dot(x_ref[...], wemb_ref[...], preferred_element_type=jnp.float32)
    y = y + bemb_ref[...]
    h0 = jax.nn.relu(_bn_rows(y, gemb_ref[...], beemb_ref[...]))
    h1_ref[...] = jnp.dot(h0, wc0_ref[...], preferred_element_type=jnp.float32)


def _mid_kernel(a0_ref, a1_ref, hlin_ref, dinv_ref, b_ref, g_ref, be_ref,
                wnext_ref, out_ref):
    d2 = dinv_ref[...] * dinv_ref[...]
    gc = a0_ref[...] + a1_ref[...] + d2 * hlin_ref[...] + b_ref[...]
    h = jax.nn.relu(_bn_rows(gc, g_ref[...], be_ref[...]))
    out_ref[...] = jnp.dot(h, wnext_ref[...], preferred_element_type=jnp.float32)


def _head_kernel(a0_ref, a1_ref, hlin_ref, dinv_ref, bc1_ref, gbn1_ref,
                 bebn1_ref, bid_ref, speed_ref, wsp_ref, bsp_ref, gsp_ref,
                 besp_ref, r0_ref, r1_ref, wrc_ref, brc_ref, grc_ref,
                 berc_ref, wrl_ref, brl_ref, w0p_ref, w0v_ref, w0r_ref,
                 b0_ref, go_ref, beo_ref, w1_ref, b1_ref, out_ref):
    d2 = dinv_ref[...] * dinv_ref[...]
    gc = a0_ref[...] + a1_ref[...] + d2 * hlin_ref[...] + bc1_ref[...]
    h2 = jax.nn.relu(_bn_rows(gc, gbn1_ref[...], bebn1_ref[...]))

    gid = lax.broadcasted_iota(jnp.int32, (G, N), 0)
    onehot = (bid_ref[...] == gid).astype(jnp.float32)
    psum = jnp.dot(onehot, h2, preferred_element_type=jnp.float32)
    counts = jnp.sum(onehot, axis=1, keepdims=True)
    pooled = psum / jnp.maximum(counts, 1.0)

    v = jax.nn.relu(_bn_rows(
        jnp.dot(speed_ref[...], wsp_ref[...], preferred_element_type=jnp.float32)
        + bsp_ref[...], gsp_ref[...], besp_ref[...]))

    zc = jnp.zeros((G, 1), jnp.float32)
    rc = brc_ref[...] * jnp.ones((G, 10), jnp.float32)
    for ci, rr in ((0, r0_ref[...]), (1, r1_ref[...])):
        zp = jnp.concatenate([zc, rr, zc], axis=1)
        for k in range(3):
            rc = rc + wrc_ref[ci:ci + 1, k:k + 1] * zp[:, k:k + 10]
    m = jnp.mean(rc)
    var = jnp.mean((rc - m) * (rc - m))
    rc = jax.nn.relu(grc_ref[...] * (rc - m) * lax.rsqrt(var + 1e-5) + berc_ref[...])
    r = jnp.dot(rc, wrl_ref[...], preferred_element_type=jnp.float32) + brl_ref[...]

    y0 = (jnp.dot(pooled, w0p_ref[...], preferred_element_type=jnp.float32)
          + jnp.dot(v, w0v_ref[...], preferred_element_type=jnp.float32)
          + jnp.dot(r, w0r_ref[...], preferred_element_type=jnp.float32)
          + b0_ref[...])
    y = jax.nn.relu(_bn_rows(y0, go_ref[...], beo_ref[...]))
    out_ref[...] = jnp.dot(y, w1_ref[...], preferred_element_type=jnp.float32) + b1_ref[...]


def _tc(fn, out_shape, *args):
    return pl.pallas_call(fn, out_shape=out_shape)(*args)


def kernel(x, edge_weight, speed, route, params, edge_index, batch_ids):
    p = params
    src = edge_index[0]
    dst = edge_index[1]

    deg_parts = _deg_call(dst, edge_weight)

    h1lin = _tc(
        _emb_kernel, jax.ShapeDtypeStruct((N, H), jnp.float32),
        x, p['W_emb'], p['b_emb'].reshape(1, F), p['g_emb'].reshape(1, F),
        p['be_emb'].reshape(1, F), p['W_c0'])

    acc1, dinv_flat = _agg_call(h1lin, src, dst, edge_weight, deg_parts)
    dinv_col = dinv_flat[:N].reshape(N, 1)

    h2lin = _tc(
        _mid_kernel, jax.ShapeDtypeStruct((N, H), jnp.float32),
        acc1[0, :N], acc1[1, :N], h1lin, dinv_col, p['b_c0'].reshape(1, H),
        p['g_bn0'].reshape(1, H), p['be_bn0'].reshape(1, H), p['W_c1'])

    acc2, _ = _agg_call(h2lin, src, dst, edge_weight, deg_parts)

    out = _tc(
        _head_kernel, jax.ShapeDtypeStruct((G, 8), jnp.float32),
        acc2[0, :N], acc2[1, :N], h2lin, dinv_col, p['b_c1'].reshape(1, H),
        p['g_bn1'].reshape(1, H), p['be_bn1'].reshape(1, H),
        batch_ids.reshape(1, N), speed, p['W_sp'], p['b_sp'].reshape(1, 4),
        p['g_sp'].reshape(1, 4), p['be_sp'].reshape(1, 4),
        route[:, :, 0], route[:, :, 1], p['W_rc'][0], p['b_rc'].reshape(1, 1),
        p['g_rc'].reshape(1, 1), p['be_rc'].reshape(1, 1), p['W_rl'],
        p['b_rl'].reshape(1, 4), p['W_o0'][:H], p['W_o0'][H:H + 4],
        p['W_o0'][H + 4:], p['b_o0'].reshape(1, 16), p['g_o'].reshape(1, 16),
        p['be_o'].reshape(1, 16), p['W_o1'], p['b_o1'].reshape(1, 8))
    return jnp.squeeze(out)


# speedup vs baseline: 46.1667x; 1.4486x over previous
import functools

import jax
import jax.numpy as jnp
from jax import lax
from jax.experimental import pallas as pl
from jax.experimental.pallas import tpu as pltpu
from jax.experimental.pallas import tpu_sc as plsc

N = 10000
NP = 10240
E = 320000
F = 128
H = 16
G = 256
NW = 32
EPW = E // NW
CH = 2000
NCH = EPW // CH
PT = NP // 16


def _deg_body(dst_hbm, w_hbm, out_hbm, hist, dst_v, w_v, degp):
    core = lax.axis_index("c")
    sid = lax.axis_index("s")
    wid = core * 16 + sid
    base = wid * EPW

    zero16 = jnp.zeros((16,), jnp.float32)

    @plsc.parallel_loop(0, 8 * NP // 16, unroll=4)
    def z(i):
        hist[pl.ds(i * 16, 16)] = zero16

    pltpu.sync_copy(dst_hbm.at[pl.ds(base, EPW)], dst_v)
    pltpu.sync_copy(w_hbm.at[pl.ds(base, EPW)], w_v)

    iota = lax.iota(jnp.int32, 16)
    laneoff = (iota & 7) * NP
    m_lo = iota < 8
    m_hi = iota >= 8

    def acc(g, _):
        d16 = dst_v[pl.ds(g * 16, 16)]
        w16 = w_v[pl.ds(g * 16, 16)]
        idx = d16 + laneoff
        plsc.addupdate_scatter(hist, [idx], w16, mask=m_lo)
        plsc.addupdate_scatter(hist, [idx], w16, mask=m_hi)
        return 0
    lax.fori_loop(0, EPW // 16, acc, 0)

    @plsc.parallel_loop(0, NP // 16, unroll=2)
    def red(i):
        s = hist[pl.ds(i * 16, 16)]
        for l in range(1, 8):
            s = s + hist[pl.ds(l * NP + i * 16, 16)]
        degp[pl.ds(i * 16, 16)] = s

    pltpu.sync_copy(degp, out_hbm.at[wid])


_deg_call = functools.partial(
    pl.kernel,
    out_type=jax.ShapeDtypeStruct((NW, NP), jnp.float32),
    scratch_types=[
        pltpu.VMEM((8 * NP,), jnp.float32),
        pltpu.VMEM((EPW,), jnp.int32),
        pltpu.VMEM((EPW,), jnp.float32),
        pltpu.VMEM((NP,), jnp.float32),
    ],
    mesh=plsc.VectorSubcoreMesh(core_axis_name="c", subcore_axis_name="s"),
    compiler_params=pltpu.CompilerParams(needs_layout_passes=False, use_tc_tiling_on_sc=False),
)(_deg_body)


def _rsqrt16(x):
    i = plsc.bitcast(x, jnp.int32)
    y = plsc.bitcast(jnp.full((16,), 0x5F3759DF, jnp.int32) - (i >> 1), jnp.float32)
    xh = x * 0.5
    for _ in range(4):
        y = y * (1.5 - xh * y * y)
    return y


def _agg_body(h_hbm, src_hbm, dst_hbm, w_hbm, degp_hbm, out_hbm, dinv_hbm,
              degs_v, dsl_v, dinv_v, src0, dst0, w0, src1, dst1, w1,
              norm_v, rows0, rows1, acc_sh, dinv_sh,
              gsem0, gsem1, ssem0, ssem1):
    core = lax.axis_index("c")
    sid = lax.axis_index("s")
    wid = core * 16 + sid
    base = wid * EPW

    pltpu.sync_copy(degp_hbm.at[:, pl.ds(sid * PT, PT)], degs_v)

    @plsc.parallel_loop(0, PT // 16, unroll=2)
    def dv(i):
        s = degs_v[0, pl.ds(i * 16, 16)]
        for l in range(1, NW):
            s = s + degs_v[l, pl.ds(i * 16, 16)]
        dsl_v[pl.ds(i * 16, 16)] = _rsqrt16(s + 1.0)
    pltpu.sync_copy(dsl_v, dinv_sh.at[pl.ds(sid * PT, PT)])

    zero16 = jnp.zeros((16,), jnp.float32)

    @plsc.parallel_loop(0, PT, unroll=4)
    def z(i):
        rows0[i, :] = zero16
    pltpu.sync_copy(rows0.at[pl.ds(0, PT)], acc_sh.at[pl.ds(sid * PT, PT)])
    plsc.subcore_barrier()
    pltpu.sync_copy(dinv_sh, dinv_v)

    @pl.when(jnp.logical_and(core == 0, sid == 0))
    def _():
        pltpu.sync_copy(dinv_v, dinv_hbm)

    bufs = ((src0, dst0, w0, rows0, gsem0, ssem0),
            (src1, dst1, w1, rows1, gsem1, ssem1))

    def stage(c):
        srcb, dstb, wb, rowsb, gsem, _ = bufs[c % 2]
        eb = base + c * CH
        pltpu.sync_copy(src_hbm.at[pl.ds(eb, CH)], srcb)
        pltpu.sync_copy(dst_hbm.at[pl.ds(eb, CH)], dstb)
        pltpu.sync_copy(w_hbm.at[pl.ds(eb, CH)], wb)
        pltpu.async_copy(h_hbm.at[srcb], rowsb, gsem)

    stage(0)
    for c in range(NCH):
        srcb, dstb, wb, rowsb, gsem, ssem = bufs[c % 2]
        pltpu.make_async_copy(h_hbm.at[srcb], rowsb, gsem).wait()
        if c + 1 < NCH:
            if c >= 1:
                pltpu.make_async_copy(
                    bufs[(c + 1) % 2][3], acc_sh.at[bufs[(c + 1) % 2][1]],
                    bufs[(c + 1) % 2][5]).wait()
            stage(c + 1)

        @plsc.parallel_loop(0, CH // 16, unroll=2)
        def nrm(j):
            s16 = srcb[pl.ds(j * 16, 16)]
            d16 = dstb[pl.ds(j * 16, 16)]
            w16 = wb[pl.ds(j * 16, 16)]
            nm = plsc.load_gather(dinv_v, [s16]) * w16 * plsc.load_gather(dinv_v, [d16])
            norm_v[pl.ds(j * 16, 16)] = nm

        @plsc.parallel_loop(0, CH // 16, unroll=2)
        def scale(j):
            for u in range(16):
                e = j * 16 + u
                nv = plsc.load_gather(norm_v, [jnp.broadcast_to(e, (16,))])
                rowsb[e, :] = rowsb[e, :] * nv

        pltpu.async_copy(rowsb, acc_sh.at[dstb], ssem, add=True)

    pltpu.make_async_copy(bufs[(NCH - 2) % 2][3],
                          acc_sh.at[bufs[(NCH - 2) % 2][1]],
                          bufs[(NCH - 2) % 2][5]).wait()
    pltpu.make_async_copy(bufs[(NCH - 1) % 2][3],
                          acc_sh.at[bufs[(NCH - 1) % 2][1]],
                          bufs[(NCH - 1) % 2][5]).wait()
    plsc.subcore_barrier()
    pltpu.sync_copy(acc_sh.at[pl.ds(sid * PT, PT)],
                    out_hbm.at[core, pl.ds(sid * PT, PT)])


_agg_call = functools.partial(
    pl.kernel,
    out_type=(jax.ShapeDtypeStruct((2, NP, H), jnp.float32),
              jax.ShapeDtypeStruct((NP,), jnp.float32)),
    scratch_types=[
        pltpu.VMEM((NW, PT), jnp.float32),
        pltpu.VMEM((PT,), jnp.float32),
        pltpu.VMEM((NP,), jnp.float32),
        pltpu.VMEM((CH,), jnp.int32),
        pltpu.VMEM((CH,), jnp.int32),
        pltpu.VMEM((CH,), jnp.float32),
        pltpu.VMEM((CH,), jnp.int32),
        pltpu.VMEM((CH,), jnp.int32),
        pltpu.VMEM((CH,), jnp.float32),
        pltpu.VMEM((CH,), jnp.float32),
        pltpu.VMEM((CH, H), jnp.float32),
        pltpu.VMEM((CH, H), jnp.float32),
        pltpu.VMEM_SHARED((NP, H), jnp.float32),
        pltpu.VMEM_SHARED((NP,), jnp.float32),
        pltpu.SemaphoreType.DMA,
        pltpu.SemaphoreType.DMA,
        pltpu.SemaphoreType.DMA,
        pltpu.SemaphoreType.DMA,
    ],
    mesh=plsc.VectorSubcoreMesh(core_axis_name="c", subcore_axis_name="s"),
    compiler_params=pltpu.CompilerParams(needs_layout_passes=False, use_tc_tiling_on_sc=False),
)(_agg_body)


def _bn_rows(y, g, b):
    m = jnp.mean(y, axis=0, keepdims=True)
    v = jnp.mean((y - m) * (y - m), axis=0, keepdims=True)
    return g * (y - m) * lax.rsqrt(v + 1e-5) + b


def _emb_kernel(x_ref, wemb_ref, bemb_ref, gemb_ref, beemb_ref, wc0_ref,
                h1_ref):
    y = jnp.dot(x_ref[...], wemb_ref[...], preferred_element_type=jnp.float32)
    y = y + bemb_ref[...]
    h0 = jax.nn.relu(_bn_rows(y, gemb_ref[...], beemb_ref[...]))
    h1_ref[...] = jnp.dot(h0, wc0_ref[...], preferred_element_type=jnp.float32)


def _mid_kernel(a0_ref, a1_ref, hlin_ref, dinv_ref, b_ref, g_ref, be_ref,
                wnext_ref, out_ref):
    d2 = dinv_ref[...] * dinv_ref[...]
    gc = a0_ref[...] + a1_ref[...] + d2 * hlin_ref[...] + b_ref[...]
    h = jax.nn.relu(_bn_rows(gc, g_ref[...], be_ref[...]))
    out_ref[...] = jnp.dot(h, wnext_ref[...], preferred_element_type=jnp.float32)


def _head_kernel(a0_ref, a1_ref, hlin_ref, dinv_ref, bc1_ref, gbn1_ref,
                 bebn1_ref, bid_ref, speed_ref, wsp_ref, bsp_ref, gsp_ref,
                 besp_ref, r0_ref, r1_ref, wrc_ref, brc_ref, grc_ref,
                 berc_ref, wrl_ref, brl_ref, w0p_ref, w0v_ref, w0r_ref,
                 b0_ref, go_ref, beo_ref, w1_ref, b1_ref, out_ref):
    d2 = dinv_ref[...] * dinv_ref[...]
    gc = a0_ref[...] + a1_ref[...] + d2 * hlin_ref[...] + bc1_ref[...]
    h2 = jax.nn.relu(_bn_rows(gc, gbn1_ref[...], bebn1_ref[...]))

    gid = lax.broadcasted_iota(jnp.int32, (G, N), 0)
    onehot = (bid_ref[...] == gid).astype(jnp.float32)
    psum = jnp.dot(onehot, h2, preferred_element_type=jnp.float32)
    counts = jnp.sum(onehot, axis=1, keepdims=True)
    pooled = psum / jnp.maximum(counts, 1.0)

    v = jax.nn.relu(_bn_rows(
        jnp.dot(speed_ref[...], wsp_ref[...], preferred_element_type=jnp.float32)
        + bsp_ref[...], gsp_ref[...], besp_ref[...]))

    zc = jnp.zeros((G, 1), jnp.float32)
    rc = brc_ref[...] * jnp.ones((G, 10), jnp.float32)
    for ci, rr in ((0, r0_ref[...]), (1, r1_ref[...])):
        zp = jnp.concatenate([zc, rr, zc], axis=1)
        for k in range(3):
            rc = rc + wrc_ref[ci:ci + 1, k:k + 1] * zp[:, k:k + 10]
    m = jnp.mean(rc)
    var = jnp.mean((rc - m) * (rc - m))
    rc = jax.nn.relu(grc_ref[...] * (rc - m) * lax.rsqrt(var + 1e-5) + berc_ref[...])
    r = jnp.dot(rc, wrl_ref[...], preferred_element_type=jnp.float32) + brl_ref[...]

    y0 = (jnp.dot(pooled, w0p_ref[...], preferred_element_type=jnp.float32)
          + jnp.dot(v, w0v_ref[...], preferred_element_type=jnp.float32)
          + jnp.dot(r, w0r_ref[...], preferred_element_type=jnp.float32)
          + b0_ref[...])
    y = jax.nn.relu(_bn_rows(y0, go_ref[...], beo_ref[...]))
    out_ref[...] = jnp.dot(y, w1_ref[...], preferred_element_type=jnp.float32) + b1_ref[...]


def _tc(fn, out_shape, *args):
    return pl.pallas_call(fn, out_shape=out_shape)(*args)


def kernel(x, edge_weight, speed, route, params, edge_index, batch_ids):
    p = params
    src = edge_index[0]
    dst = edge_index[1]

    deg_parts = _deg_call(dst, edge_weight)

    h1lin = _tc(
        _emb_kernel, jax.ShapeDtypeStruct((N, H), jnp.float32),
        x, p['W_emb'], p['b_emb'].reshape(1, F), p['g_emb'].reshape(1, F),
        p['be_emb'].reshape(1, F), p['W_c0'])

    acc1, dinv_flat = _agg_call(h1lin, src, dst, edge_weight, deg_parts)
    dinv_col = dinv_flat[:N].reshape(N, 1)

    h2lin = _tc(
        _mid_kernel, jax.ShapeDtypeStruct((N, H), jnp.float32),
        acc1[0, :N], acc1[1, :N], h1lin, dinv_col, p['b_c0'].reshape(1, H),
        p['g_bn0'].reshape(1, H), p['be_bn0'].reshape(1, H), p['W_c1'])

    acc2, _ = _agg_call(h2lin, src, dst, edge_weight, deg_parts)

    out = _tc(
        _head_kernel, jax.ShapeDtypeStruct((G, 8), jnp.float32),
        acc2[0, :N], acc2[1, :N], h2lin, dinv_col, p['b_c1'].reshape(1, H),
        p['g_bn1'].reshape(1, H), p['be_bn1'].reshape(1, H),
        batch_ids.reshape(1, N), speed, p['W_sp'], p['b_sp'].reshape(1, 4),
        p['g_sp'].reshape(1, 4), p['be_sp'].reshape(1, 4),
        route[:, :, 0], route[:, :, 1], p['W_rc'][0], p['b_rc'].reshape(1, 1),
        p['g_rc'].reshape(1, 1), p['be_rc'].reshape(1, 1), p['W_rl'],
        p['b_rl'].reshape(1, 4), p['W_o0'][:H], p['W_o0'][H:H + 4],
        p['W_o0'][H + 4:], p['b_o0'].reshape(1, 16), p['g_o'].reshape(1, 16),
        p['be_o'].reshape(1, 16), p['W_o1'], p['b_o1'].reshape(1, 8))
    return jnp.squeeze(out)


# speedup vs baseline: 46.6083x; 1.0096x over previous
import functools

import jax
import jax.numpy as jnp
from jax import lax
from jax.experimental import pallas as pl
from jax.experimental.pallas import tpu as pltpu
from jax.experimental.pallas import tpu_sc as plsc

N = 10000
NP = 10240
E = 320000
F = 128
H = 16
G = 256
NW = 32
EPW = E // NW
CH = 2000
NCH = EPW // CH
PT = NP // 16


def _deg_body(dst_hbm, w_hbm, out_hbm, hist, dst_v, w_v, degp):
    core = lax.axis_index("c")
    sid = lax.axis_index("s")
    wid = core * 16 + sid
    base = wid * EPW

    zero16 = jnp.zeros((16,), jnp.float32)

    @plsc.parallel_loop(0, 8 * NP // 16, unroll=4)
    def z(i):
        hist[pl.ds(i * 16, 16)] = zero16

    pltpu.sync_copy(dst_hbm.at[pl.ds(base, EPW)], dst_v)
    pltpu.sync_copy(w_hbm.at[pl.ds(base, EPW)], w_v)

    iota = lax.iota(jnp.int32, 16)
    laneoff = (iota & 7) * NP
    m_lo = iota < 8
    m_hi = iota >= 8

    def acc(g, _):
        d16 = dst_v[pl.ds(g * 16, 16)]
        w16 = w_v[pl.ds(g * 16, 16)]
        idx = d16 + laneoff
        plsc.addupdate_scatter(hist, [idx], w16, mask=m_lo)
        plsc.addupdate_scatter(hist, [idx], w16, mask=m_hi)
        return 0
    lax.fori_loop(0, EPW // 16, acc, 0)

    @plsc.parallel_loop(0, NP // 16, unroll=2)
    def red(i):
        s = hist[pl.ds(i * 16, 16)]
        for l in range(1, 8):
            s = s + hist[pl.ds(l * NP + i * 16, 16)]
        degp[pl.ds(i * 16, 16)] = s

    pltpu.sync_copy(degp, out_hbm.at[wid])


_deg_call = functools.partial(
    pl.kernel,
    out_type=jax.ShapeDtypeStruct((NW, NP), jnp.float32),
    scratch_types=[
        pltpu.VMEM((8 * NP,), jnp.float32),
        pltpu.VMEM((EPW,), jnp.int32),
        pltpu.VMEM((EPW,), jnp.float32),
        pltpu.VMEM((NP,), jnp.float32),
    ],
    mesh=plsc.VectorSubcoreMesh(core_axis_name="c", subcore_axis_name="s"),
    compiler_params=pltpu.CompilerParams(needs_layout_passes=False, use_tc_tiling_on_sc=False),
)(_deg_body)


def _rsqrt16(x):
    i = plsc.bitcast(x, jnp.int32)
    y = plsc.bitcast(jnp.full((16,), 0x5F3759DF, jnp.int32) - (i >> 1), jnp.float32)
    xh = x * 0.5
    for _ in range(4):
        y = y * (1.5 - xh * y * y)
    return y


def _agg_body(h_hbm, src_hbm, dst_hbm, w_hbm, degp_hbm, out_hbm, dinv_hbm,
              degs_v, dsl_v, dinv_v, src0, dst0, w0, src1, dst1, w1,
              norm_v, rows0, rows1, acc_sh, dinv_sh,
              gsem0, gsem1, ssem0, ssem1):
    core = lax.axis_index("c")
    sid = lax.axis_index("s")
    wid = core * 16 + sid
    base = wid * EPW

    pltpu.sync_copy(degp_hbm.at[:, pl.ds(sid * PT, PT)], degs_v)

    @plsc.parallel_loop(0, PT // 16, unroll=2)
    def dv(i):
        s = degs_v[0, pl.ds(i * 16, 16)]
        for l in range(1, NW):
            s = s + degs_v[l, pl.ds(i * 16, 16)]
        dsl_v[pl.ds(i * 16, 16)] = _rsqrt16(s + 1.0)
    pltpu.sync_copy(dsl_v, dinv_sh.at[pl.ds(sid * PT, PT)])

    zero16 = jnp.zeros((16,), jnp.float32)

    @plsc.parallel_loop(0, PT, unroll=4)
    def z(i):
        rows0[i, :] = zero16
    pltpu.sync_copy(rows0.at[pl.ds(0, PT)], acc_sh.at[pl.ds(sid * PT, PT)])
    plsc.subcore_barrier()
    pltpu.sync_copy(dinv_sh, dinv_v)

    @pl.when(jnp.logical_and(core == 0, sid == 0))
    def _():
        pltpu.sync_copy(dinv_v, dinv_hbm)

    bufs = ((src0, dst0, w0, rows0, gsem0, ssem0),
            (src1, dst1, w1, rows1, gsem1, ssem1))

    def stage(c):
        srcb, dstb, wb, rowsb, gsem, _ = bufs[c % 2]
        eb = base + c * CH
        pltpu.sync_copy(src_hbm.at[pl.ds(eb, CH)], srcb)
        pltpu.sync_copy(dst_hbm.at[pl.ds(eb, CH)], dstb)
        pltpu.sync_copy(w_hbm.at[pl.ds(eb, CH)], wb)
        pltpu.async_copy(h_hbm.at[srcb], rowsb, gsem)

    stage(0)
    for c in range(NCH):
        srcb, dstb, wb, rowsb, gsem, ssem = bufs[c % 2]
        pltpu.make_async_copy(h_hbm.at[srcb], rowsb, gsem).wait()
        if c + 1 < NCH:
            if c >= 1:
                pltpu.make_async_copy(
                    bufs[(c + 1) % 2][3], acc_sh.at[bufs[(c + 1) % 2][1]],
                    bufs[(c + 1) % 2][5]).wait()
            stage(c + 1)

        @plsc.parallel_loop(0, CH // 16, unroll=4)
        def nrm(j):
            s16 = srcb[pl.ds(j * 16, 16)]
            d16 = dstb[pl.ds(j * 16, 16)]
            w16 = wb[pl.ds(j * 16, 16)]
            nm = plsc.load_gather(dinv_v, [s16]) * w16 * plsc.load_gather(dinv_v, [d16])
            norm_v[pl.ds(j * 16, 16)] = nm

        @plsc.parallel_loop(0, CH // 16, unroll=4)
        def scale(j):
            for u in range(16):
                e = j * 16 + u
                nv = plsc.load_gather(norm_v, [jnp.broadcast_to(e, (16,))])
                rowsb[e, :] = rowsb[e, :] * nv

        pltpu.async_copy(rowsb, acc_sh.at[dstb], ssem, add=True)

    pltpu.make_async_copy(bufs[(NCH - 2) % 2][3],
                          acc_sh.at[bufs[(NCH - 2) % 2][1]],
                          bufs[(NCH - 2) % 2][5]).wait()
    pltpu.make_async_copy(bufs[(NCH - 1) % 2][3],
                          acc_sh.at[bufs[(NCH - 1) % 2][1]],
                          bufs[(NCH - 1) % 2][5]).wait()
    plsc.subcore_barrier()
    pltpu.sync_copy(acc_sh.at[pl.ds(sid * PT, PT)],
                    out_hbm.at[core, pl.ds(sid * PT, PT)])


_agg_call = functools.partial(
    pl.kernel,
    out_type=(jax.ShapeDtypeStruct((2, NP, H), jnp.float32),
              jax.ShapeDtypeStruct((NP,), jnp.float32)),
    scratch_types=[
        pltpu.VMEM((NW, PT), jnp.float32),
        pltpu.VMEM((PT,), jnp.float32),
        pltpu.VMEM((NP,), jnp.float32),
        pltpu.VMEM((CH,), jnp.int32),
        pltpu.VMEM((CH,), jnp.int32),
        pltpu.VMEM((CH,), jnp.float32),
        pltpu.VMEM((CH,), jnp.int32),
        pltpu.VMEM((CH,), jnp.int32),
        pltpu.VMEM((CH,), jnp.float32),
        pltpu.VMEM((CH,), jnp.float32),
        pltpu.VMEM((CH, H), jnp.float32),
        pltpu.VMEM((CH, H), jnp.float32),
        pltpu.VMEM_SHARED((NP, H), jnp.float32),
        pltpu.VMEM_SHARED((NP,), jnp.float32),
        pltpu.SemaphoreType.DMA,
        pltpu.SemaphoreType.DMA,
        pltpu.SemaphoreType.DMA,
        pltpu.SemaphoreType.DMA,
    ],
    mesh=plsc.VectorSubcoreMesh(core_axis_name="c", subcore_axis_name="s"),
    compiler_params=pltpu.CompilerParams(needs_layout_passes=False, use_tc_tiling_on_sc=False),
)(_agg_body)


def _bn_rows(y, g, b):
    m = jnp.mean(y, axis=0, keepdims=True)
    v = jnp.mean((y - m) * (y - m), axis=0, keepdims=True)
    return g * (y - m) * lax.rsqrt(v + 1e-5) + b


def _emb_kernel(x_ref, wemb_ref, bemb_ref, gemb_ref, beemb_ref, wc0_ref,
                h1_ref):
    y = jnp.dot(x_ref[...], wemb_ref[...], preferred_element_type=jnp.float32)
    y = y + bemb_ref[...]
    h0 = jax.nn.relu(_bn_rows(y, gemb_ref[...], beemb_ref[...]))
    h1_ref[...] = jnp.dot(h0, wc0_ref[...], preferred_element_type=jnp.float32)


def _mid_kernel(a0_ref, a1_ref, hlin_ref, dinv_ref, b_ref, g_ref, be_ref,
                wnext_ref, out_ref):
    d2 = dinv_ref[...] * dinv_ref[...]
    gc = a0_ref[...] + a1_ref[...] + d2 * hlin_ref[...] + b_ref[...]
    h = jax.nn.relu(_bn_rows(gc, g_ref[...], be_ref[...]))
    out_ref[...] = jnp.dot(h, wnext_ref[...], preferred_element_type=jnp.float32)


def _head_kernel(a0_ref, a1_ref, hlin_ref, dinv_ref, bc1_ref, gbn1_ref,
                 bebn1_ref, bid_ref, speed_ref, wsp_ref, bsp_ref, gsp_ref,
                 besp_ref, r0_ref, r1_ref, wrc_ref, brc_ref, grc_ref,
                 berc_ref, wrl_ref, brl_ref, w0p_ref, w0v_ref, w0r_ref,
                 b0_ref, go_ref, beo_ref, w1_ref, b1_ref, out_ref):
    d2 = dinv_ref[...] * dinv_ref[...]
    gc = a0_ref[...] + a1_ref[...] + d2 * hlin_ref[...] + bc1_ref[...]
    h2 = jax.nn.relu(_bn_rows(gc, gbn1_ref[...], bebn1_ref[...]))

    gid = lax.broadcasted_iota(jnp.int32, (G, N), 0)
    onehot = (bid_ref[...] == gid).astype(jnp.float32)
    psum = jnp.dot(onehot, h2, preferred_element_type=jnp.float32)
    counts = jnp.sum(onehot, axis=1, keepdims=True)
    pooled = psum / jnp.maximum(counts, 1.0)

    v = jax.nn.relu(_bn_rows(
        jnp.dot(speed_ref[...], wsp_ref[...], preferred_element_type=jnp.float32)
        + bsp_ref[...], gsp_ref[...], besp_ref[...]))

    zc = jnp.zeros((G, 1), jnp.float32)
    rc = brc_ref[...] * jnp.ones((G, 10), jnp.float32)
    for ci, rr in ((0, r0_ref[...]), (1, r1_ref[...])):
        zp = jnp.concatenate([zc, rr, zc], axis=1)
        for k in range(3):
            rc = rc + wrc_ref[ci:ci + 1, k:k + 1] * zp[:, k:k + 10]
    m = jnp.mean(rc)
    var = jnp.mean((rc - m) * (rc - m))
    rc = jax.nn.relu(grc_ref[...] * (rc - m) * lax.rsqrt(var + 1e-5) + berc_ref[...])
    r = jnp.dot(rc, wrl_ref[...], preferred_element_type=jnp.float32) + brl_ref[...]

    y0 = (jnp.dot(pooled, w0p_ref[...], preferred_element_type=jnp.float32)
          + jnp.dot(v, w0v_ref[...], preferred_element_type=jnp.float32)
          + jnp.dot(r, w0r_ref[...], preferred_element_type=jnp.float32)
          + b0_ref[...])
    y = jax.nn.relu(_bn_rows(y0, go_ref[...], beo_ref[...]))
    out_ref[...] = jnp.dot(y, w1_ref[...], preferred_element_type=jnp.float32) + b1_ref[...]


def _tc(fn, out_shape, *args):
    return pl.pallas_call(fn, out_shape=out_shape)(*args)


def kernel(x, edge_weight, speed, route, params, edge_index, batch_ids):
    p = params
    src = edge_index[0]
    dst = edge_index[1]

    deg_parts = _deg_call(dst, edge_weight)

    h1lin = _tc(
        _emb_kernel, jax.ShapeDtypeStruct((N, H), jnp.float32),
        x, p['W_emb'], p['b_emb'].reshape(1, F), p['g_emb'].reshape(1, F),
        p['be_emb'].reshape(1, F), p['W_c0'])

    acc1, dinv_flat = _agg_call(h1lin, src, dst, edge_weight, deg_parts)
    dinv_col = dinv_flat[:N].reshape(N, 1)

    h2lin = _tc(
        _mid_kernel, jax.ShapeDtypeStruct((N, H), jnp.float32),
        acc1[0, :N], acc1[1, :N], h1lin, dinv_col, p['b_c0'].reshape(1, H),
        p['g_bn0'].reshape(1, H), p['be_bn0'].reshape(1, H), p['W_c1'])

    acc2, _ = _agg_call(h2lin, src, dst, edge_weight, deg_parts)

    out = _tc(
        _head_kernel, jax.ShapeDtypeStruct((G, 8), jnp.float32),
        acc2[0, :N], acc2[1, :N], h2lin, dinv_col, p['b_c1'].reshape(1, H),
        p['g_bn1'].reshape(1, H), p['be_bn1'].reshape(1, H),
        batch_ids.reshape(1, N), speed, p['W_sp'], p['b_sp'].reshape(1, 4),
        p['g_sp'].reshape(1, 4), p['be_sp'].reshape(1, 4),
        route[:, :, 0], route[:, :, 1], p['W_rc'][0], p['b_rc'].reshape(1, 1),
        p['g_rc'].reshape(1, 1), p['be_rc'].reshape(1, 1), p['W_rl'],
        p['b_rl'].reshape(1, 4), p['W_o0'][:H], p['W_o0'][H:H + 4],
        p['W_o0'][H + 4:], p['b_o0'].reshape(1, 16), p['g_o'].reshape(1, 16),
        p['be_o'].reshape(1, 16), p['W_o1'], p['b_o1'].reshape(1, 8))
    return jnp.squeeze(out)


# speedup vs baseline: 47.7260x; 1.0240x over previous
import functools

import jax
import jax.numpy as jnp
from jax import lax
from jax.experimental import pallas as pl
from jax.experimental.pallas import tpu as pltpu
from jax.experimental.pallas import tpu_sc as plsc

N = 10000
NP = 10240
E = 320000
F = 128
H = 16
G = 256
NW = 32
EPW = E // NW
CH = 2000
NCH = EPW // CH
PT = NP // 16


def _deg_body(dst_hbm, w_hbm, out_hbm, hist, dst_v, w_v, degp):
    core = lax.axis_index("c")
    sid = lax.axis_index("s")
    wid = core * 16 + sid
    base = wid * EPW

    zero16 = jnp.zeros((16,), jnp.float32)

    @plsc.parallel_loop(0, 8 * NP // 16, unroll=4)
    def z(i):
        hist[pl.ds(i * 16, 16)] = zero16

    pltpu.sync_copy(dst_hbm.at[pl.ds(base, EPW)], dst_v)
    pltpu.sync_copy(w_hbm.at[pl.ds(base, EPW)], w_v)

    iota = lax.iota(jnp.int32, 16)
    laneoff = (iota & 7) * NP
    m_lo = iota < 8
    m_hi = iota >= 8

    def acc(g, _):
        d16 = dst_v[pl.ds(g * 16, 16)]
        w16 = w_v[pl.ds(g * 16, 16)]
        idx = d16 + laneoff
        plsc.addupdate_scatter(hist, [idx], w16, mask=m_lo)
        plsc.addupdate_scatter(hist, [idx], w16, mask=m_hi)
        return 0
    lax.fori_loop(0, EPW // 16, acc, 0)

    @plsc.parallel_loop(0, NP // 16, unroll=2)
    def red(i):
        s = hist[pl.ds(i * 16, 16)]
        for l in range(1, 8):
            s = s + hist[pl.ds(l * NP + i * 16, 16)]
        degp[pl.ds(i * 16, 16)] = s

    pltpu.sync_copy(degp, out_hbm.at[wid])


_deg_call = functools.partial(
    pl.kernel,
    out_type=jax.ShapeDtypeStruct((NW, NP), jnp.float32),
    scratch_types=[
        pltpu.VMEM((8 * NP,), jnp.float32),
        pltpu.VMEM((EPW,), jnp.int32),
        pltpu.VMEM((EPW,), jnp.float32),
        pltpu.VMEM((NP,), jnp.float32),
    ],
    mesh=plsc.VectorSubcoreMesh(core_axis_name="c", subcore_axis_name="s"),
    compiler_params=pltpu.CompilerParams(needs_layout_passes=False, use_tc_tiling_on_sc=False),
)(_deg_body)


def _rsqrt16(x):
    i = plsc.bitcast(x, jnp.int32)
    y = plsc.bitcast(jnp.full((16,), 0x5F3759DF, jnp.int32) - (i >> 1), jnp.float32)
    xh = x * 0.5
    for _ in range(4):
        y = y * (1.5 - xh * y * y)
    return y


def _agg_body(h_hbm, src_hbm, dst_hbm, w_hbm, degp_hbm, out_hbm, dinv_hbm,
              degs_v, dsl_v, dinv_v, src0, dst0, w0, src1, dst1, w1,
              norm_v, rows0, rows1, acc_sh, dinv_sh,
              gsem0, gsem1, ssem0, ssem1):
    core = lax.axis_index("c")
    sid = lax.axis_index("s")
    wid = core * 16 + sid
    base = wid * EPW

    pltpu.sync_copy(degp_hbm.at[:, pl.ds(sid * PT, PT)], degs_v)

    @plsc.parallel_loop(0, PT // 16, unroll=2)
    def dv(i):
        s = degs_v[0, pl.ds(i * 16, 16)]
        for l in range(1, NW):
            s = s + degs_v[l, pl.ds(i * 16, 16)]
        dsl_v[pl.ds(i * 16, 16)] = _rsqrt16(s + 1.0)
    pltpu.sync_copy(dsl_v, dinv_sh.at[pl.ds(sid * PT, PT)])

    zero16 = jnp.zeros((16,), jnp.float32)

    @plsc.parallel_loop(0, PT, unroll=4)
    def z(i):
        rows0[i, :] = zero16
    pltpu.sync_copy(rows0.at[pl.ds(0, PT)], acc_sh.at[pl.ds(sid * PT, PT)])
    plsc.subcore_barrier()
    pltpu.sync_copy(dinv_sh, dinv_v)

    @pl.when(jnp.logical_and(core == 0, sid == 0))
    def _():
        pltpu.sync_copy(dinv_v, dinv_hbm)

    bufs = ((src0, dst0, w0, rows0, gsem0, ssem0),
            (src1, dst1, w1, rows1, gsem1, ssem1))

    def stage(c):
        srcb, dstb, wb, rowsb, gsem, _ = bufs[c % 2]
        eb = base + c * CH
        pltpu.sync_copy(src_hbm.at[pl.ds(eb, CH)], srcb)
        pltpu.sync_copy(dst_hbm.at[pl.ds(eb, CH)], dstb)
        pltpu.sync_copy(w_hbm.at[pl.ds(eb, CH)], wb)
        pltpu.async_copy(h_hbm.at[srcb], rowsb, gsem)

    stage(0)
    for c in range(NCH):
        srcb, dstb, wb, rowsb, gsem, ssem = bufs[c % 2]
        pltpu.make_async_copy(h_hbm.at[srcb], rowsb, gsem).wait()
        if c + 1 < NCH:
            if c >= 1:
                pltpu.make_async_copy(
                    bufs[(c + 1) % 2][3], acc_sh.at[bufs[(c + 1) % 2][1]],
                    bufs[(c + 1) % 2][5]).wait()
            stage(c + 1)

        @plsc.parallel_loop(0, CH // 16, unroll=4)
        def nrm(j):
            s16 = srcb[pl.ds(j * 16, 16)]
            d16 = dstb[pl.ds(j * 16, 16)]
            w16 = wb[pl.ds(j * 16, 16)]
            nm = plsc.load_gather(dinv_v, [s16]) * w16 * plsc.load_gather(dinv_v, [d16])
            norm_v[pl.ds(j * 16, 16)] = nm

        @plsc.parallel_loop(0, CH // 16, unroll=4)
        def scale(j):
            for u in range(16):
                e = j * 16 + u
                nv = plsc.load_gather(norm_v, [jnp.broadcast_to(e, (16,))])
                rowsb[e, :] = rowsb[e, :] * nv

        pltpu.async_copy(rowsb, acc_sh.at[dstb], ssem, add=True)

    pltpu.make_async_copy(bufs[(NCH - 2) % 2][3],
                          acc_sh.at[bufs[(NCH - 2) % 2][1]],
                          bufs[(NCH - 2) % 2][5]).wait()
    pltpu.make_async_copy(bufs[(NCH - 1) % 2][3],
                          acc_sh.at[bufs[(NCH - 1) % 2][1]],
                          bufs[(NCH - 1) % 2][5]).wait()
    plsc.subcore_barrier()
    pltpu.sync_copy(acc_sh.at[pl.ds(sid * PT, PT)],
                    out_hbm.at[core, pl.ds(sid * PT, PT)])


_agg_call = functools.partial(
    pl.kernel,
    out_type=(jax.ShapeDtypeStruct((2, NP, H), jnp.float32),
              jax.ShapeDtypeStruct((NP,), jnp.float32)),
    scratch_types=[
        pltpu.VMEM((NW, PT), jnp.float32),
        pltpu.VMEM((PT,), jnp.float32),
        pltpu.VMEM((NP,), jnp.float32),
        pltpu.VMEM((CH,), jnp.int32),
        pltpu.VMEM((CH,), jnp.int32),
        pltpu.VMEM((CH,), jnp.float32),
        pltpu.VMEM((CH,), jnp.int32),
        pltpu.VMEM((CH,), jnp.int32),
        pltpu.VMEM((CH,), jnp.float32),
        pltpu.VMEM((CH,), jnp.float32),
        pltpu.VMEM((CH, H), jnp.float32),
        pltpu.VMEM((CH, H), jnp.float32),
        pltpu.VMEM_SHARED((NP, H), jnp.float32),
        pltpu.VMEM_SHARED((NP,), jnp.float32),
        pltpu.SemaphoreType.DMA,
        pltpu.SemaphoreType.DMA,
        pltpu.SemaphoreType.DMA,
        pltpu.SemaphoreType.DMA,
    ],
    mesh=plsc.VectorSubcoreMesh(core_axis_name="c", subcore_axis_name="s"),
    compiler_params=pltpu.CompilerParams(needs_layout_passes=False, use_tc_tiling_on_sc=False),
)(_agg_body)


def _agg2_body(h_hbm, src_hbm, dst_hbm, w_hbm, out_hbm,
               src0, dst0, w0, src1, dst1, w1, rows0, rows1, acc_sh,
               gsem0, gsem1, ssem0, ssem1):
    core = lax.axis_index("c")
    sid = lax.axis_index("s")
    wid = core * 16 + sid
    base = wid * EPW

    zero16 = jnp.zeros((16,), jnp.float32)

    @plsc.parallel_loop(0, PT, unroll=4)
    def z(i):
        rows0[i, :] = zero16
    pltpu.sync_copy(rows0.at[pl.ds(0, PT)], acc_sh.at[pl.ds(sid * PT, PT)])
    plsc.subcore_barrier()

    bufs = ((src0, dst0, w0, rows0, gsem0, ssem0),
            (src1, dst1, w1, rows1, gsem1, ssem1))

    def stage(c):
        srcb, dstb, wb, rowsb, gsem, _ = bufs[c % 2]
        eb = base + c * CH
        pltpu.sync_copy(src_hbm.at[pl.ds(eb, CH)], srcb)
        pltpu.sync_copy(dst_hbm.at[pl.ds(eb, CH)], dstb)
        pltpu.sync_copy(w_hbm.at[pl.ds(eb, CH)], wb)
        pltpu.async_copy(h_hbm.at[srcb], rowsb, gsem)

    stage(0)
    for c in range(NCH):
        srcb, dstb, wb, rowsb, gsem, ssem = bufs[c % 2]
        pltpu.make_async_copy(h_hbm.at[srcb], rowsb, gsem).wait()
        if c + 1 < NCH:
            if c >= 1:
                pltpu.make_async_copy(
                    bufs[(c + 1) % 2][3], acc_sh.at[bufs[(c + 1) % 2][1]],
                    bufs[(c + 1) % 2][5]).wait()
            stage(c + 1)

        @plsc.parallel_loop(0, CH // 16, unroll=4)
        def scale(j):
            for u in range(16):
                e = j * 16 + u
                nv = plsc.load_gather(wb, [jnp.broadcast_to(e, (16,))])
                rowsb[e, :] = rowsb[e, :] * nv

        pltpu.async_copy(rowsb, acc_sh.at[dstb], ssem, add=True)

    pltpu.make_async_copy(bufs[(NCH - 2) % 2][3],
                          acc_sh.at[bufs[(NCH - 2) % 2][1]],
                          bufs[(NCH - 2) % 2][5]).wait()
    pltpu.make_async_copy(bufs[(NCH - 1) % 2][3],
                          acc_sh.at[bufs[(NCH - 1) % 2][1]],
                          bufs[(NCH - 1) % 2][5]).wait()
    plsc.subcore_barrier()
    pltpu.sync_copy(acc_sh.at[pl.ds(sid * PT, PT)],
                    out_hbm.at[core, pl.ds(sid * PT, PT)])


_agg2_call = functools.partial(
    pl.kernel,
    out_type=jax.ShapeDtypeStruct((2, NP, H), jnp.float32),
    scratch_types=[
        pltpu.VMEM((CH,), jnp.int32),
        pltpu.VMEM((CH,), jnp.int32),
        pltpu.VMEM((CH,), jnp.float32),
        pltpu.VMEM((CH,), jnp.int32),
        pltpu.VMEM((CH,), jnp.int32),
        pltpu.VMEM((CH,), jnp.float32),
        pltpu.VMEM((CH, H), jnp.float32),
        pltpu.VMEM((CH, H), jnp.float32),
        pltpu.VMEM_SHARED((NP, H), jnp.float32),
        pltpu.SemaphoreType.DMA,
        pltpu.SemaphoreType.DMA,
        pltpu.SemaphoreType.DMA,
        pltpu.SemaphoreType.DMA,
    ],
    mesh=plsc.VectorSubcoreMesh(core_axis_name="c", subcore_axis_name="s"),
    compiler_params=pltpu.CompilerParams(needs_layout_passes=False, use_tc_tiling_on_sc=False),
)(_agg2_body)


def _bn_rows(y, g, b):
    m = jnp.mean(y, axis=0, keepdims=True)
    v = jnp.mean((y - m) * (y - m), axis=0, keepdims=True)
    return g * (y - m) * lax.rsqrt(v + 1e-5) + b


def _emb_kernel(x_ref, wemb_ref, bemb_ref, gemb_ref, beemb_ref, wc0_ref,
                h1_ref):
    y = jnp.dot(x_ref[...], wemb_ref[...], preferred_element_type=jnp.float32)
    y = y + bemb_ref[...]
    h0 = jax.nn.relu(_bn_rows(y, gemb_ref[...], beemb_ref[...]))
    h1_ref[...] = jnp.dot(h0, wc0_ref[...], preferred_element_type=jnp.float32)


def _mid_kernel(a0_ref, a1_ref, hlin_ref, dinv_ref, b_ref, g_ref, be_ref,
                wnext_ref, out_ref):
    d = dinv_ref[...]
    gc = a0_ref[...] + a1_ref[...] + d * d * hlin_ref[...] + b_ref[...]
    h = jax.nn.relu(_bn_rows(gc, g_ref[...], be_ref[...]))
    out_ref[...] = d * jnp.dot(h, wnext_ref[...], preferred_element_type=jnp.float32)


def _head_kernel(a0_ref, a1_ref, hlin_ref, dinv_ref, bc1_ref, gbn1_ref,
                 bebn1_ref, bid_ref, speed_ref, wsp_ref, bsp_ref, gsp_ref,
                 besp_ref, r0_ref, r1_ref, wrc_ref, brc_ref, grc_ref,
                 berc_ref, wrl_ref, brl_ref, w0p_ref, w0v_ref, w0r_ref,
                 b0_ref, go_ref, beo_ref, w1_ref, b1_ref, out_ref):
    gc = dinv_ref[...] * (a0_ref[...] + a1_ref[...] + hlin_ref[...]) + bc1_ref[...]
    h2 = jax.nn.relu(_bn_rows(gc, gbn1_ref[...], bebn1_ref[...]))

    gid = lax.broadcasted_iota(jnp.int32, (G, N), 0)
    onehot = (bid_ref[...] == gid).astype(jnp.float32)
    psum = jnp.dot(onehot, h2, preferred_element_type=jnp.float32)
    counts = jnp.sum(onehot, axis=1, keepdims=True)
    pooled = psum / jnp.maximum(counts, 1.0)

    v = jax.nn.relu(_bn_rows(
        jnp.dot(speed_ref[...], wsp_ref[...], preferred_element_type=jnp.float32)
        + bsp_ref[...], gsp_ref[...], besp_ref[...]))

    zc = jnp.zeros((G, 1), jnp.float32)
    rc = brc_ref[...] * jnp.ones((G, 10), jnp.float32)
    for ci, rr in ((0, r0_ref[...]), (1, r1_ref[...])):
        zp = jnp.concatenate([zc, rr, zc], axis=1)
        for k in range(3):
            rc = rc + wrc_ref[ci:ci + 1, k:k + 1] * zp[:, k:k + 10]
    m = jnp.mean(rc)
    var = jnp.mean((rc - m) * (rc - m))
    rc = jax.nn.relu(grc_ref[...] * (rc - m) * lax.rsqrt(var + 1e-5) + berc_ref[...])
    r = jnp.dot(rc, wrl_ref[...], preferred_element_type=jnp.float32) + brl_ref[...]

    y0 = (jnp.dot(pooled, w0p_ref[...], preferred_element_type=jnp.float32)
          + jnp.dot(v, w0v_ref[...], preferred_element_type=jnp.float32)
          + jnp.dot(r, w0r_ref[...], preferred_element_type=jnp.float32)
          + b0_ref[...])
    y = jax.nn.relu(_bn_rows(y0, go_ref[...], beo_ref[...]))
    out_ref[...] = jnp.dot(y, w1_ref[...], preferred_element_type=jnp.float32) + b1_ref[...]


def _tc(fn, out_shape, *args):
    return pl.pallas_call(fn, out_shape=out_shape)(*args)


def kernel(x, edge_weight, speed, route, params, edge_index, batch_ids):
    p = params
    src = edge_index[0]
    dst = edge_index[1]

    deg_parts = _deg_call(dst, edge_weight)

    h1lin = _tc(
        _emb_kernel, jax.ShapeDtypeStruct((N, H), jnp.float32),
        x, p['W_emb'], p['b_emb'].reshape(1, F), p['g_emb'].reshape(1, F),
        p['be_emb'].reshape(1, F), p['W_c0'])

    acc1, dinv_flat = _agg_call(h1lin, src, dst, edge_weight, deg_parts)
    dinv_col = dinv_flat[:N].reshape(N, 1)

    h2lin = _tc(
        _mid_kernel, jax.ShapeDtypeStruct((N, H), jnp.float32),
        acc1[0, :N], acc1[1, :N], h1lin, dinv_col, p['b_c0'].reshape(1, H),
        p['g_bn0'].reshape(1, H), p['be_bn0'].reshape(1, H), p['W_c1'])

    acc2 = _agg2_call(h2lin, src, dst, edge_weight)

    out = _tc(
        _head_kernel, jax.ShapeDtypeStruct((G, 8), jnp.float32),
        acc2[0, :N], acc2[1, :N], h2lin, dinv_col, p['b_c1'].reshape(1, H),
        p['g_bn1'].reshape(1, H), p['be_bn1'].reshape(1, H),
        batch_ids.reshape(1, N), speed, p['W_sp'], p['b_sp'].reshape(1, 4),
        p['g_sp'].reshape(1, 4), p['be_sp'].reshape(1, 4),
        route[:, :, 0], route[:, :, 1], p['W_rc'][0], p['b_rc'].reshape(1, 1),
        p['g_rc'].reshape(1, 1), p['be_rc'].reshape(1, 1), p['W_rl'],
        p['b_rl'].reshape(1, 4), p['W_o0'][:H], p['W_o0'][H:H + 4],
        p['W_o0'][H + 4:], p['b_o0'].reshape(1, 16), p['g_o'].reshape(1, 16),
        p['be_o'].reshape(1, 16), p['W_o1'], p['b_o1'].reshape(1, 8))
    return jnp.squeeze(out)
